# Initial kernel scaffold; baseline (speedup 1.0000x reference)
import functools

import jax
import jax.numpy as jnp
from jax.experimental import pallas as pl
from jax.experimental.pallas import tpu as pltpu

B, S, H, E, F = 2, 2048, 768, 8, 3072
T = B * S
TB = 512
NT = T // TB


def _router_body(x_ref, wg_ref, gs_ref, eid_ref, g_ref, loss_ref):
    x = x_ref[...]
    wg = wg_ref[...]
    logits = jax.lax.dot_general(
        x, wg, (((1,), (1,)), ((), ())),
        preferred_element_type=jnp.float32)
    m = jnp.max(logits, axis=-1, keepdims=True)
    ex = jnp.exp(logits - m)
    gs = ex / jnp.sum(ex, axis=-1, keepdims=True)
    gs_ref[...] = gs
    top = jnp.max(gs, axis=-1, keepdims=True)
    lanes = jax.lax.broadcasted_iota(jnp.int32, (T, E), 1)
    eid = jnp.min(jnp.where(gs == top, lanes, E), axis=-1, keepdims=True)
    eid_ref[...] = eid
    g_ref[...] = top / (top + 1e-08)
    onehot = (lanes == eid).astype(jnp.float32)
    usage = jnp.sum(onehot, axis=0, keepdims=True) / T
    probs = jnp.sum(gs, axis=0, keepdims=True) / T
    loss_ref[...] = (E * jnp.sum(probs * usage)).reshape(1, 1)


def _ffn_body(x_ref, w1_ref, b1_ref, w2_ref, b2_ref, eid_ref, g_ref, out_ref):
    e = pl.program_id(1)
    x = x_ref[...]
    h = jax.lax.dot_general(
        x, w1_ref[0], (((1,), (1,)), ((), ())),
        preferred_element_type=jnp.float32)
    h = jnp.maximum(h + b1_ref[...], 0.0)
    y = jax.lax.dot_general(
        h, w2_ref[0], (((1,), (1,)), ((), ())),
        preferred_element_type=jnp.float32)
    y = y + b2_ref[...]
    w = jnp.where(eid_ref[...] == e, g_ref[...], 0.0)
    contrib = w * y

    @pl.when(e == 0)
    def _init():
        out_ref[...] = contrib

    @pl.when(e != 0)
    def _acc():
        out_ref[...] += contrib


@jax.jit
def kernel(x, Wg, W1, b1, W2, b2):
    x_flat = x.reshape(T, H)

    gs, eid, g, loss = pl.pallas_call(
        _router_body,
        out_shape=(
            jax.ShapeDtypeStruct((T, E), jnp.float32),
            jax.ShapeDtypeStruct((T, 1), jnp.int32),
            jax.ShapeDtypeStruct((T, 1), jnp.float32),
            jax.ShapeDtypeStruct((1, 1), jnp.float32),
        ),
    )(x_flat, Wg)

    out = pl.pallas_call(
        _ffn_body,
        grid=(NT, E),
        in_specs=[
            pl.BlockSpec((TB, H), lambda t, e: (t, 0)),
            pl.BlockSpec((1, F, H), lambda t, e: (e, 0, 0)),
            pl.BlockSpec((1, F), lambda t, e: (e, 0)),
            pl.BlockSpec((1, H, F), lambda t, e: (e, 0, 0)),
            pl.BlockSpec((1, H), lambda t, e: (e, 0)),
            pl.BlockSpec((TB, 1), lambda t, e: (t, 0)),
            pl.BlockSpec((TB, 1), lambda t, e: (t, 0)),
        ],
        out_specs=pl.BlockSpec((TB, H), lambda t, e: (t, 0)),
        out_shape=jax.ShapeDtypeStruct((T, H), jnp.float32),
        compiler_params=pltpu.CompilerParams(
            dimension_semantics=("parallel", "arbitrary"),
        ),
    )(x_flat, W1, b1, W2, b2, eid, g)

    return out.reshape(B, S, H), gs.reshape(B, S, E), loss.reshape(())


# speedup vs baseline: 1.0657x; 1.0657x over previous
import functools

import jax
import jax.numpy as jnp
from jax.experimental import pallas as pl
from jax.experimental.pallas import tpu as pltpu

B, S, H, E, F = 2, 2048, 768, 8, 3072
T = B * S
TB = 512
NT = T // TB


def _router_body(x_ref, wg_ref, gs_ref, eid_ref, g_ref, loss_ref):
    x = x_ref[...]
    wg = wg_ref[...]
    logits = jax.lax.dot_general(
        x, wg, (((1,), (1,)), ((), ())),
        preferred_element_type=jnp.float32)
    m = jnp.max(logits, axis=-1, keepdims=True)
    ex = jnp.exp(logits - m)
    gs = ex / jnp.sum(ex, axis=-1, keepdims=True)
    gs_ref[...] = gs
    top = jnp.max(gs, axis=-1, keepdims=True)
    lanes = jax.lax.broadcasted_iota(jnp.int32, (T, E), 1)
    eid = jnp.min(jnp.where(gs == top, lanes, E), axis=-1, keepdims=True)
    eid_ref[...] = eid
    g_ref[...] = top / (top + 1e-08)
    onehot = (lanes == eid).astype(jnp.float32)
    usage = jnp.sum(onehot, axis=0, keepdims=True) / T
    probs = jnp.sum(gs, axis=0, keepdims=True) / T
    loss_ref[...] = (E * jnp.sum(probs * usage)).reshape(1, 1)


def _ffn_body(x_ref, w1_ref, b1_ref, w2_ref, b2_ref, eid_ref, g_ref, out_ref):
    e = pl.program_id(1)
    x = x_ref[...]
    h = jax.lax.dot_general(
        x, w1_ref[0], (((1,), (1,)), ((), ())),
        preferred_element_type=jnp.float32)
    h = jnp.maximum(h + b1_ref[0], 0.0)
    y = jax.lax.dot_general(
        h, w2_ref[0], (((1,), (1,)), ((), ())),
        preferred_element_type=jnp.float32)
    y = y + b2_ref[0]
    w = jnp.where(eid_ref[...] == e, g_ref[...], 0.0)
    contrib = w * y

    @pl.when(e == 0)
    def _init():
        out_ref[...] = contrib

    @pl.when(e != 0)
    def _acc():
        out_ref[...] += contrib


@jax.jit
def kernel(x, Wg, W1, b1, W2, b2):
    x_flat = x.reshape(T, H)

    gs, eid, g, loss = pl.pallas_call(
        _router_body,
        out_shape=(
            jax.ShapeDtypeStruct((T, E), jnp.float32),
            jax.ShapeDtypeStruct((T, 1), jnp.int32),
            jax.ShapeDtypeStruct((T, 1), jnp.float32),
            jax.ShapeDtypeStruct((1, 1), jnp.float32),
        ),
    )(x_flat, Wg)

    out = pl.pallas_call(
        _ffn_body,
        grid=(NT, E),
        in_specs=[
            pl.BlockSpec((TB, H), lambda t, e: (t, 0)),
            pl.BlockSpec((1, F, H), lambda t, e: (e, 0, 0)),
            pl.BlockSpec((1, 1, F), lambda t, e: (e, 0, 0)),
            pl.BlockSpec((1, H, F), lambda t, e: (e, 0, 0)),
            pl.BlockSpec((1, 1, H), lambda t, e: (e, 0, 0)),
            pl.BlockSpec((TB, 1), lambda t, e: (t, 0)),
            pl.BlockSpec((TB, 1), lambda t, e: (t, 0)),
        ],
        out_specs=pl.BlockSpec((TB, H), lambda t, e: (t, 0)),
        out_shape=jax.ShapeDtypeStruct((T, H), jnp.float32),
        compiler_params=pltpu.CompilerParams(
            dimension_semantics=("parallel", "arbitrary"),
        ),
    )(x_flat, W1, b1.reshape(E, 1, F), W2, b2.reshape(E, 1, H), eid, g)

    return out.reshape(B, S, H), gs.reshape(B, S, E), loss.reshape(())


# speedup vs baseline: 1.6185x; 1.5187x over previous
import functools

import jax
import jax.numpy as jnp
from jax import lax
from jax.experimental import pallas as pl
from jax.experimental.pallas import tpu as pltpu
from jax.experimental.pallas import tpu_sc as plsc

B, S, H, E, F = 2, 2048, 768, 8, 3072
T = B * S
TB = 256
NB = T // TB + E
NP = NB * TB
NBP = 32
CH = 512

NC, NS = 2, 16
NPC = NP // NC
RPT = NPC // NS
TPT = T // (NC * NS)
GCH = 64


def _router_body(x_ref, wg_ref, gs_ref, pos_ref, g_ref, be_ref, bv_ref,
                 loss_ref):
    x = x_ref[...]
    wg = wg_ref[...]
    logits = lax.dot_general(x, wg, (((1,), (1,)), ((), ())),
                             preferred_element_type=jnp.float32)
    m = jnp.max(logits, axis=-1, keepdims=True)
    ex = jnp.exp(logits - m)
    gs = ex / jnp.sum(ex, axis=-1, keepdims=True)
    gs_ref[...] = gs
    top = jnp.max(gs, axis=-1, keepdims=True)
    lanes = lax.broadcasted_iota(jnp.int32, (T, E), 1)
    eid = jnp.min(jnp.where(gs == top, lanes, E), axis=-1, keepdims=True)
    g_ref[...] = top / (top + 1e-08)
    onehot = (lanes == eid).astype(jnp.float32)
    usage = jnp.sum(onehot, axis=0, keepdims=True) / T
    probs = jnp.sum(gs, axis=0, keepdims=True) / T
    loss_ref[...] = (E * jnp.sum(probs * usage)).reshape(1, 1)

    tri = (lax.broadcasted_iota(jnp.int32, (CH, CH), 0)
           >= lax.broadcasted_iota(jnp.int32, (CH, CH), 1)).astype(jnp.float32)
    run = jnp.zeros((1, E), jnp.float32)
    rank_chunks = []
    for ci in range(T // CH):
        oh = lax.slice(onehot, (ci * CH, 0), ((ci + 1) * CH, E))
        rank = lax.dot_general(tri, oh, (((1,), (0,)), ((), ())),
                               preferred_element_type=jnp.float32) + run
        run = lax.slice(rank, (CH - 1, 0), (CH, E))
        rank_chunks.append(rank)
    ranks = jnp.concatenate(rank_chunks, axis=0)
    counts = run
    padded = jnp.ceil(counts / TB) * TB
    lt8 = (lax.broadcasted_iota(jnp.int32, (E, E), 0)
           < lax.broadcasted_iota(jnp.int32, (E, E), 1)).astype(jnp.float32)
    base = lax.dot_general(padded, lt8, (((1,), (0,)), ((), ())),
                           preferred_element_type=jnp.float32)
    rank_sel = jnp.sum(onehot * ranks, axis=-1, keepdims=True)
    base_sel = jnp.sum(onehot * base, axis=-1, keepdims=True)
    pos_ref[...] = (base_sel + rank_sel - 1.0).astype(jnp.int32)

    ends = base + padded
    bs = lax.broadcasted_iota(jnp.int32, (NBP, 1), 0).astype(jnp.float32) * TB
    ge = (bs >= ends).astype(jnp.float32)
    raw = jnp.sum(ge, axis=-1, keepdims=True)
    total = jnp.sum(padded)
    be_ref[...] = jnp.minimum(raw, E - 1.0).astype(jnp.int32)
    bv_ref[...] = (bs < total).astype(jnp.int32)


def _ffn_body(be_ref, bv_ref, x_ref, w1_ref, b1_ref, w2_ref, b2_ref, g_ref,
              out_ref):
    b = pl.program_id(0)

    @pl.when(bv_ref[b] != 0)
    def _compute():
        h = lax.dot_general(x_ref[...], w1_ref[0], (((1,), (1,)), ((), ())),
                            preferred_element_type=jnp.float32)
        h = jnp.maximum(h + b1_ref[0], 0.0)
        y = lax.dot_general(h, w2_ref[0], (((1,), (1,)), ((), ())),
                            preferred_element_type=jnp.float32)
        out_ref[...] = g_ref[...] * (y + b2_ref[0])


def _sc_dispatch_body(pos_hbm, g_hbm, x_hbm, xs_hbm, gsort_hbm, tok_hbm,
                      pos_v, g_v, tok_v, gs_v, idx64, rows, sem):
    c = lax.axis_index("c")
    s = lax.axis_index("s")
    half = c * NPC

    @pl.when(s == 0)
    def _bookkeep():
        pltpu.sync_copy(pos_hbm, pos_v)
        pltpu.sync_copy(g_hbm, g_v)

        def zini(i, carry):
            tok_v[pl.ds(i * 16, 16)] = jnp.zeros((16,), jnp.int32)
            return carry
        lax.fori_loop(0, NP // 16, zini, 0)

        def scat(i, carry):
            pc = pos_v[pl.ds(i * 16, 16)]
            plsc.store_scatter(tok_v, [pc], lax.iota(jnp.int32, 16) + i * 16)
            return carry
        lax.fori_loop(0, T // 16, scat, 0)

        def gath(i, carry):
            tk = tok_v[pl.ds(i * 16, 16)]
            gs_v[pl.ds(i * 16, 16)] = plsc.load_gather(g_v, [tk])
            return carry
        lax.fori_loop(0, NP // 16, gath, 0)

        pltpu.sync_copy(gs_v.at[pl.ds(half, NPC)], gsort_hbm.at[pl.ds(half, NPC)])
        pltpu.sync_copy(tok_v.at[pl.ds(half, NPC)], tok_hbm.at[pl.ds(half, NPC)])

    plsc.subcore_barrier()

    tbase = half + s * RPT

    def gloop(j, carry):
        o = tbase + j * GCH
        pltpu.sync_copy(tok_hbm.at[pl.ds(o, GCH)], idx64)
        pltpu.async_copy(x_hbm.at[idx64], rows, sem).wait()
        pltpu.sync_copy(rows, xs_hbm.at[pl.ds(o, GCH)])
        return carry
    lax.fori_loop(0, RPT // GCH, gloop, 0)


def _sc_combine_body(pos_hbm, ys_hbm, out_hbm, idx64, rows, sem):
    c = lax.axis_index("c")
    s = lax.axis_index("s")
    tbase = (c * NS + s) * TPT

    def gloop(j, carry):
        o = tbase + j * GCH
        pltpu.sync_copy(pos_hbm.at[pl.ds(o, GCH)], idx64)
        pltpu.async_copy(ys_hbm.at[idx64], rows, sem).wait()
        pltpu.sync_copy(rows, out_hbm.at[pl.ds(o, GCH)])
        return carry
    lax.fori_loop(0, TPT // GCH, gloop, 0)


@functools.lru_cache(maxsize=1)
def _sc_kernels():
    mesh = plsc.VectorSubcoreMesh(core_axis_name="c", subcore_axis_name="s")
    dispatch = pl.kernel(
        _sc_dispatch_body,
        out_type=(
            jax.ShapeDtypeStruct((NP, H), jnp.float32),
            jax.ShapeDtypeStruct((NP,), jnp.float32),
            jax.ShapeDtypeStruct((NP,), jnp.int32),
        ),
        mesh=mesh,
        scratch_types=(
            pltpu.VMEM((T,), jnp.int32),
            pltpu.VMEM((T,), jnp.float32),
            pltpu.VMEM((NP,), jnp.int32),
            pltpu.VMEM((NP,), jnp.float32),
            pltpu.VMEM((GCH,), jnp.int32),
            pltpu.VMEM((GCH, H), jnp.float32),
            pltpu.SemaphoreType.DMA,
        ),
        compiler_params=pltpu.CompilerParams(needs_layout_passes=False),
    )
    combine = pl.kernel(
        _sc_combine_body,
        out_type=jax.ShapeDtypeStruct((T, H), jnp.float32),
        mesh=mesh,
        scratch_types=(
            pltpu.VMEM((GCH,), jnp.int32),
            pltpu.VMEM((GCH, H), jnp.float32),
            pltpu.SemaphoreType.DMA,
        ),
        compiler_params=pltpu.CompilerParams(needs_layout_passes=False),
    )
    return dispatch, combine


@jax.jit
def kernel(x, Wg, W1, b1, W2, b2):
    x_flat = x.reshape(T, H)

    gs, pos2, g2, be2, bv2, loss = pl.pallas_call(
        _router_body,
        out_shape=(
            jax.ShapeDtypeStruct((T, E), jnp.float32),
            jax.ShapeDtypeStruct((T, 1), jnp.int32),
            jax.ShapeDtypeStruct((T, 1), jnp.float32),
            jax.ShapeDtypeStruct((NBP, 1), jnp.int32),
            jax.ShapeDtypeStruct((NBP, 1), jnp.int32),
            jax.ShapeDtypeStruct((1, 1), jnp.float32),
        ),
    )(x_flat, Wg)
    pos = pos2.reshape(T)
    g = g2.reshape(T)

    _sc_dispatch, _sc_combine = _sc_kernels()
    x_sorted, g_sorted, _tok = _sc_dispatch(pos, g, x_flat)

    y_sorted = pl.pallas_call(
        _ffn_body,
        grid_spec=pltpu.PrefetchScalarGridSpec(
            num_scalar_prefetch=2,
            grid=(NB,),
            in_specs=[
                pl.BlockSpec((TB, H), lambda b, be, bv: (b, 0)),
                pl.BlockSpec((1, F, H), lambda b, be, bv: (be[b], 0, 0)),
                pl.BlockSpec((1, 1, F), lambda b, be, bv: (be[b], 0, 0)),
                pl.BlockSpec((1, H, F), lambda b, be, bv: (be[b], 0, 0)),
                pl.BlockSpec((1, 1, H), lambda b, be, bv: (be[b], 0, 0)),
                pl.BlockSpec((TB, 1), lambda b, be, bv: (b, 0)),
            ],
            out_specs=pl.BlockSpec((TB, H), lambda b, be, bv: (b, 0)),
        ),
        out_shape=jax.ShapeDtypeStruct((NP, H), jnp.float32),
        compiler_params=pltpu.CompilerParams(
            dimension_semantics=("arbitrary",),
        ),
    )(be2.reshape(NBP), bv2.reshape(NBP), x_sorted, W1,
      b1.reshape(E, 1, F), W2, b2.reshape(E, 1, H),
      g_sorted.reshape(NP, 1))

    out_flat = _sc_combine(pos, y_sorted)

    return out_flat.reshape(B, S, H), gs.reshape(B, S, E), loss.reshape(())
